# baseline (speedup 1.0000x reference)
import dataclasses
import functools

import jax
import jax.numpy as jnp
from jax import lax
from jax.experimental import pallas as pl
from jax.experimental.pallas import tpu as pltpu
from jax.experimental.pallas import tpu_sc as plsc

N_NODES = 10000
N_EDGES = 320000
D = 128

NC = 2
NS = 16
NW = NC * NS
CHUNK = 128
NCH0 = 160
EPT0 = CHUNK * NCH0
E_PAD = NS * EPT0
RPT = 632
NPAD = NS * RPT


def _linear_body(x_ref, w_ref, b_ref, o_ref):
    o_ref[...] = (
        jnp.dot(x_ref[...], w_ref[...], preferred_element_type=jnp.float32)
        + b_ref[...]
    )


def _linear(x, W, b):
    return pl.pallas_call(
        _linear_body,
        grid=(5,),
        in_specs=[
            pl.BlockSpec((2000, D), lambda i: (i, 0)),
            pl.BlockSpec((D, D), lambda i: (0, 0)),
            pl.BlockSpec((1, D), lambda i: (0, 0)),
        ],
        out_specs=pl.BlockSpec((2000, D), lambda i: (i, 0)),
        out_shape=jax.ShapeDtypeStruct((N_NODES, D), jnp.float32),
    )(x, W, b.reshape(1, D))


def _sc_compiler_params():
    cp = pltpu.CompilerParams()
    if "needs_layout_passes" in pltpu.CompilerParams.__dataclass_fields__:
        cp = dataclasses.replace(cp, needs_layout_passes=False)
    return cp


def _spmm_sc(h, src, dst, w):
    mesh = plsc.VectorSubcoreMesh(core_axis_name="c", subcore_axis_name="s")

    @functools.partial(
        pl.kernel,
        compiler_params=_sc_compiler_params(),
        out_type=jax.ShapeDtypeStruct((N_NODES, D), jnp.float32),
        mesh=mesh,
        scratch_types=[
            pltpu.VMEM((2, CHUNK), jnp.int32),
            pltpu.VMEM((2, CHUNK), jnp.int32),
            pltpu.VMEM((2, CHUNK), jnp.float32),
            pltpu.VMEM((CHUNK, D), jnp.float32),
            pltpu.VMEM((CHUNK, D), jnp.float32),
            pltpu.VMEM_SHARED((NPAD, D), jnp.float32),
            pltpu.SemaphoreType.DMA,
            pltpu.SemaphoreType.DMA,
            pltpu.SemaphoreType.DMA,
            pltpu.SemaphoreType.DMA,
            pltpu.SemaphoreType.DMA,
            pltpu.SemaphoreType.DMA,
        ],
    )
    def k(h_hbm, src_hbm, dst_hbm, w_hbm, out0,
          src_v, dst_v, w_v, rows0, rows1, acc,
          gsem0, gsem1, dwsem0, dwsem1, ssem0, ssem1):
        cid = lax.axis_index("c")
        sid = lax.axis_index("s")
        row0 = sid * RPT

        @pl.when(cid == 0)
        def _():
            @pl.loop(0, CHUNK)
            def _(r):
                for j in range(D // 16):
                    rows1[r, pl.ds(j * 16, 16)] = jnp.zeros((16,), jnp.float32)

            for kk in range(RPT // CHUNK):
                pltpu.sync_copy(rows1, acc.at[pl.ds(row0 + kk * CHUNK, CHUNK)])
            _rem = RPT % CHUNK
            pltpu.sync_copy(
                rows1.at[pl.ds(0, _rem)],
                acc.at[pl.ds(row0 + (RPT // CHUNK) * CHUNK, _rem)],
            )
        plsc.subcore_barrier()

        rbufs = (rows0, rows1)
        gsems = (gsem0, gsem1)
        dwsems = (dwsem0, dwsem1)
        ssems = (ssem0, ssem1)

        def src_issue(base, ci, b):
            off = base + ci * CHUNK
            pltpu.async_copy(src_hbm.at[pl.ds(off, CHUNK)], src_v.at[b], ssems[b])

        def src_wait(b):
            pltpu.make_async_copy(
                src_hbm.at[pl.ds(0, CHUNK)], src_v.at[b], ssems[b]
            ).wait()

        def dw_issue(base, ci, b):
            off = base + ci * CHUNK
            pltpu.async_copy(dst_hbm.at[pl.ds(off, CHUNK)], dst_v.at[b], dwsems[b])
            pltpu.async_copy(w_hbm.at[pl.ds(off, CHUNK)], w_v.at[b], dwsems[b])

        def dw_wait(b):
            pltpu.make_async_copy(
                dst_hbm.at[pl.ds(0, CHUNK)], dst_v.at[b], dwsems[b]
            ).wait()
            pltpu.make_async_copy(
                w_hbm.at[pl.ds(0, CHUNK)], w_v.at[b], dwsems[b]
            ).wait()

        def g_issue(b):
            pltpu.async_copy(h_hbm.at[src_v.at[b]], rbufs[b], gsems[b])

        def g_wait(b):
            pltpu.make_async_copy(
                h_hbm.at[src_v.at[b]], rbufs[b], gsems[b]
            ).wait()

        def scale_and_scatter(b):
            rows = rbufs[b]

            @pl.loop(0, CHUNK)
            def _(e):
                ws = plsc.load_gather(
                    w_v, [jnp.full((16,), b, jnp.int32),
                          jnp.full((16,), e, jnp.int32)]
                )
                for j in range(D // 16):
                    sl = (e, pl.ds(j * 16, 16))
                    rows[sl] = rows[sl] * ws

            pltpu.sync_copy(rows, acc.at[dst_v.at[b]], add=True)

        def half(base, nch, ci, b):
            g_wait(b)
            @pl.when(ci + 2 < nch)
            def _():
                src_issue(base, ci + 2, b)
            dw_wait(b)
            scale_and_scatter(b)
            @pl.when(ci + 2 < nch)
            def _():
                dw_issue(base, ci + 2, b)
                src_wait(b)
                g_issue(b)

        def run_core(base, nch):
            src_issue(base, 0, 0)
            src_issue(base, 1, 1)
            dw_issue(base, 0, 0)
            dw_issue(base, 1, 1)
            src_wait(0)
            g_issue(0)
            src_wait(1)
            g_issue(1)

            @pl.loop(0, nch, step=2)
            def _(ci):
                half(base, nch, ci, 0)
                half(base, nch, ci + 1, 1)

        @pl.when(cid == 0)
        def _():
            run_core(sid * EPT0, NCH0)

        plsc.subcore_barrier()

        @pl.when(jnp.logical_and(cid == 0, sid < 15))
        def _():
            pltpu.sync_copy(acc.at[pl.ds(row0, RPT)], out0.at[pl.ds(row0, RPT)])

        @pl.when(jnp.logical_and(cid == 0, sid == 15))
        def _():
            pltpu.sync_copy(
                acc.at[pl.ds(row0, N_NODES - 15 * RPT)],
                out0.at[pl.ds(row0, N_NODES - 15 * RPT)],
            )

    return k(h, src, dst, w)


def kernel(x, edge_index, edge_weight, W_ll, b_ll, W_self, b_self):
    h = _linear(x, W_ll, b_ll)
    pad = E_PAD - N_EDGES
    src = jnp.pad(edge_index[0].astype(jnp.int32), (0, pad))
    dst = jnp.concatenate(
        [edge_index[1].astype(jnp.int32), jnp.arange(pad, dtype=jnp.int32)]
    )
    w = jnp.pad(edge_weight.astype(jnp.float32), (0, pad))
    return _spmm_sc(h, src, dst, w)


# speedup vs baseline: 1.9479x; 1.9479x over previous
import dataclasses
import functools

import jax
import jax.numpy as jnp
from jax import lax
from jax.experimental import pallas as pl
from jax.experimental.pallas import tpu as pltpu
from jax.experimental.pallas import tpu_sc as plsc

N_NODES = 10000
N_EDGES = 320000
D = 128

NC = 2
NS = 16
NW = NC * NS
CHUNK = 128
NCH0 = 160
EPT0 = CHUNK * NCH0
E_PAD = NS * EPT0
RPT = 632
NPAD = NS * RPT


def _linear_body(x_ref, w_ref, b_ref, o_ref):
    o_ref[...] = (
        jnp.dot(x_ref[...], w_ref[...], preferred_element_type=jnp.float32)
        + b_ref[...]
    )


def _linear(x, W, b):
    return pl.pallas_call(
        _linear_body,
        grid=(5,),
        in_specs=[
            pl.BlockSpec((2000, D), lambda i: (i, 0)),
            pl.BlockSpec((D, D), lambda i: (0, 0)),
            pl.BlockSpec((1, D), lambda i: (0, 0)),
        ],
        out_specs=pl.BlockSpec((2000, D), lambda i: (i, 0)),
        out_shape=jax.ShapeDtypeStruct((N_NODES, D), jnp.float32),
    )(x, W, b.reshape(1, D))


def _sc_compiler_params():
    cp = pltpu.CompilerParams()
    if "needs_layout_passes" in pltpu.CompilerParams.__dataclass_fields__:
        cp = dataclasses.replace(cp, needs_layout_passes=False)
    return cp


def _spmm_sc(h, src, dst, w):
    mesh = plsc.VectorSubcoreMesh(core_axis_name="c", subcore_axis_name="s")

    @functools.partial(
        pl.kernel,
        compiler_params=_sc_compiler_params(),
        out_type=jax.ShapeDtypeStruct((N_NODES, D), jnp.float32),
        mesh=mesh,
        scratch_types=[
            pltpu.VMEM((2, CHUNK), jnp.int32),
            pltpu.VMEM((2, CHUNK), jnp.int32),
            pltpu.VMEM((2, CHUNK), jnp.float32),
            pltpu.VMEM((CHUNK, D), jnp.float32),
            pltpu.VMEM((CHUNK, D), jnp.float32),
            pltpu.VMEM_SHARED((NPAD, D), jnp.float32),
            pltpu.SemaphoreType.DMA,
            pltpu.SemaphoreType.DMA,
            pltpu.SemaphoreType.DMA,
            pltpu.SemaphoreType.DMA,
            pltpu.SemaphoreType.DMA,
            pltpu.SemaphoreType.DMA,
        ],
    )
    def k(h_hbm, src_hbm, dst_hbm, w_hbm, out0,
          src_v, dst_v, w_v, rows0, rows1, acc,
          gsem0, gsem1, dwsem0, dwsem1, ssem0, ssem1):
        cid = lax.axis_index("c")
        sid = lax.axis_index("s")
        row0 = sid * RPT

        @pl.when(cid == 0)
        def _():
            @pl.loop(0, CHUNK)
            def _(r):
                for j in range(D // 16):
                    rows1[r, pl.ds(j * 16, 16)] = jnp.zeros((16,), jnp.float32)

            for kk in range(RPT // CHUNK):
                pltpu.sync_copy(rows1, acc.at[pl.ds(row0 + kk * CHUNK, CHUNK)])
            _rem = RPT % CHUNK
            pltpu.sync_copy(
                rows1.at[pl.ds(0, _rem)],
                acc.at[pl.ds(row0 + (RPT // CHUNK) * CHUNK, _rem)],
            )
        plsc.subcore_barrier()

        rbufs = (rows0, rows1)
        gsems = (gsem0, gsem1)
        dwsems = (dwsem0, dwsem1)
        ssems = (ssem0, ssem1)

        def src_issue(base, ci, b):
            off = base + ci * CHUNK
            pltpu.async_copy(src_hbm.at[pl.ds(off, CHUNK)], src_v.at[b], ssems[b])

        def src_wait(b):
            pltpu.make_async_copy(
                src_hbm.at[pl.ds(0, CHUNK)], src_v.at[b], ssems[b]
            ).wait()

        def dw_issue(base, ci, b):
            off = base + ci * CHUNK
            pltpu.async_copy(dst_hbm.at[pl.ds(off, CHUNK)], dst_v.at[b], dwsems[b])
            pltpu.async_copy(w_hbm.at[pl.ds(off, CHUNK)], w_v.at[b], dwsems[b])

        def dw_wait(b):
            pltpu.make_async_copy(
                dst_hbm.at[pl.ds(0, CHUNK)], dst_v.at[b], dwsems[b]
            ).wait()
            pltpu.make_async_copy(
                w_hbm.at[pl.ds(0, CHUNK)], w_v.at[b], dwsems[b]
            ).wait()

        def g_issue(b):
            pltpu.async_copy(h_hbm.at[src_v.at[b]], rbufs[b], gsems[b])

        def g_wait(b):
            pltpu.make_async_copy(
                h_hbm.at[src_v.at[b]], rbufs[b], gsems[b]
            ).wait()

        def scale_and_scatter(b):
            rows = rbufs[b]

            @pl.loop(0, CHUNK)
            def _(e):
                ws = plsc.load_gather(
                    w_v, [jnp.full((16,), b, jnp.int32),
                          jnp.full((16,), e, jnp.int32)]
                )
                for j in range(D // 16):
                    sl = (e, pl.ds(j * 16, 16))
                    rows[sl] = rows[sl] * ws

            pltpu.sync_copy(rows, acc.at[dst_v.at[b]], add=True)

        def half(base, nch, ci, b):
            g_wait(b)
            @pl.when(ci + 2 < nch)
            def _():
                src_issue(base, ci + 2, b)
            dw_wait(b)
            scale_and_scatter(b)
            @pl.when(ci + 2 < nch)
            def _():
                dw_issue(base, ci + 2, b)
                src_wait(b)
                g_issue(b)

        def run_core(base, nch):
            src_issue(base, 0, 0)
            src_issue(base, 1, 1)
            dw_issue(base, 0, 0)
            dw_issue(base, 1, 1)
            src_wait(0)
            g_issue(0)
            src_wait(1)
            g_issue(1)

            @pl.loop(0, nch, step=2)
            def _(ci):
                half(base, nch, ci, 0)
                half(base, nch, ci + 1, 1)

        @pl.when(cid == 0)
        def _():
            run_core(sid * EPT0, NCH0)

        plsc.subcore_barrier()

        @pl.when(jnp.logical_and(cid == 0, sid < 15))
        def _():
            pltpu.sync_copy(acc.at[pl.ds(row0, RPT)], out0.at[pl.ds(row0, RPT)])

        @pl.when(jnp.logical_and(cid == 0, sid == 15))
        def _():
            pltpu.sync_copy(
                acc.at[pl.ds(row0, N_NODES - 15 * RPT)],
                out0.at[pl.ds(row0, N_NODES - 15 * RPT)],
            )

    return k(h, src, dst, w)


def kernel(x, edge_index, edge_weight, W_ll, b_ll, W_self, b_self):
    h = _linear(x, W_ll, b_ll)
    pad = E_PAD - N_EDGES
    src = jnp.concatenate(
        [edge_index[0].astype(jnp.int32), jnp.arange(pad, dtype=jnp.int32)]
    )
    dst = jnp.concatenate(
        [edge_index[1].astype(jnp.int32), jnp.arange(pad, dtype=jnp.int32)]
    )
    w = jnp.pad(edge_weight.astype(jnp.float32), (0, pad))
    return _spmm_sc(h, src, dst, w)


# speedup vs baseline: 3.2836x; 1.6857x over previous
import dataclasses
import functools

import jax
import jax.numpy as jnp
from jax import lax
from jax.experimental import pallas as pl
from jax.experimental.pallas import tpu as pltpu
from jax.experimental.pallas import tpu_sc as plsc

N_NODES = 10000
N_EDGES = 320000
D = 128

NC = 2
NS = 16
NW = NC * NS
CHUNK = 128
NCH0 = 80
NCH1 = 80
EPT0 = CHUNK * NCH0
EPT1 = CHUNK * NCH1
SC0_TOT = NS * EPT0
E_PAD = NS * (EPT0 + EPT1)
RPT = 632
NPAD = NS * RPT


def _linear_body(x_ref, w_ref, b_ref, o_ref):
    o_ref[...] = (
        jnp.dot(x_ref[...], w_ref[...], preferred_element_type=jnp.float32)
        + b_ref[...]
    )


def _linear(x, W, b):
    return pl.pallas_call(
        _linear_body,
        grid=(5,),
        in_specs=[
            pl.BlockSpec((2000, D), lambda i: (i, 0)),
            pl.BlockSpec((D, D), lambda i: (0, 0)),
            pl.BlockSpec((1, D), lambda i: (0, 0)),
        ],
        out_specs=pl.BlockSpec((2000, D), lambda i: (i, 0)),
        out_shape=jax.ShapeDtypeStruct((N_NODES, D), jnp.float32),
    )(x, W, b.reshape(1, D))


def _add_body(a_ref, b_ref, o_ref):
    o_ref[...] = a_ref[...] + b_ref[...]


def _add(a, b):
    return pl.pallas_call(
        _add_body,
        grid=(5,),
        in_specs=[
            pl.BlockSpec((2000, D), lambda i: (i, 0)),
            pl.BlockSpec((2000, D), lambda i: (i, 0)),
        ],
        out_specs=pl.BlockSpec((2000, D), lambda i: (i, 0)),
        out_shape=jax.ShapeDtypeStruct((N_NODES, D), jnp.float32),
    )(a, b)


def _sc_compiler_params():
    cp = pltpu.CompilerParams()
    if "needs_layout_passes" in pltpu.CompilerParams.__dataclass_fields__:
        cp = dataclasses.replace(cp, needs_layout_passes=False)
    return cp


def _spmm_sc(h, src, dst, w):
    mesh = plsc.VectorSubcoreMesh(core_axis_name="c", subcore_axis_name="s")

    @functools.partial(
        pl.kernel,
        compiler_params=_sc_compiler_params(),
        out_type=[
            jax.ShapeDtypeStruct((NPAD, D), jnp.float32),
            jax.ShapeDtypeStruct((NPAD, D), jnp.float32),
        ],
        mesh=mesh,
        scratch_types=[
            pltpu.VMEM((2, CHUNK), jnp.int32),
            pltpu.VMEM((2, CHUNK), jnp.int32),
            pltpu.VMEM((2, CHUNK), jnp.float32),
            pltpu.VMEM((CHUNK, D), jnp.float32),
            pltpu.VMEM((CHUNK, D), jnp.float32),
            pltpu.VMEM_SHARED((NPAD, D), jnp.float32),
            pltpu.SemaphoreType.DMA,
            pltpu.SemaphoreType.DMA,
            pltpu.SemaphoreType.DMA,
            pltpu.SemaphoreType.DMA,
            pltpu.SemaphoreType.DMA,
            pltpu.SemaphoreType.DMA,
        ],
    )
    def k(h_hbm, src_hbm, dst_hbm, w_hbm, out0, out1,
          src_v, dst_v, w_v, rows0, rows1, acc,
          gsem0, gsem1, dwsem0, dwsem1, ssem0, ssem1):
        cid = lax.axis_index("c")
        sid = lax.axis_index("s")
        row0 = sid * RPT

        @pl.loop(0, CHUNK)
        def _(r):
            for j in range(D // 16):
                rows1[r, pl.ds(j * 16, 16)] = jnp.zeros((16,), jnp.float32)

        for kk in range(RPT // CHUNK):
            pltpu.sync_copy(rows1, acc.at[pl.ds(row0 + kk * CHUNK, CHUNK)])
        _rem = RPT % CHUNK
        pltpu.sync_copy(
            rows1.at[pl.ds(0, _rem)],
            acc.at[pl.ds(row0 + (RPT // CHUNK) * CHUNK, _rem)],
        )
        plsc.subcore_barrier()

        rbufs = (rows0, rows1)
        gsems = (gsem0, gsem1)
        dwsems = (dwsem0, dwsem1)
        ssems = (ssem0, ssem1)

        def src_issue(base, ci, b):
            off = base + ci * CHUNK
            pltpu.async_copy(src_hbm.at[pl.ds(off, CHUNK)], src_v.at[b], ssems[b])

        def src_wait(b):
            pltpu.make_async_copy(
                src_hbm.at[pl.ds(0, CHUNK)], src_v.at[b], ssems[b]
            ).wait()

        def dw_issue(base, ci, b):
            off = base + ci * CHUNK
            pltpu.async_copy(dst_hbm.at[pl.ds(off, CHUNK)], dst_v.at[b], dwsems[b])
            pltpu.async_copy(w_hbm.at[pl.ds(off, CHUNK)], w_v.at[b], dwsems[b])

        def dw_wait(b):
            pltpu.make_async_copy(
                dst_hbm.at[pl.ds(0, CHUNK)], dst_v.at[b], dwsems[b]
            ).wait()
            pltpu.make_async_copy(
                w_hbm.at[pl.ds(0, CHUNK)], w_v.at[b], dwsems[b]
            ).wait()

        def g_issue(b):
            pltpu.async_copy(h_hbm.at[src_v.at[b]], rbufs[b], gsems[b])

        def g_wait(b):
            pltpu.make_async_copy(
                h_hbm.at[src_v.at[b]], rbufs[b], gsems[b]
            ).wait()

        def scale_and_scatter(b):
            rows = rbufs[b]

            @pl.loop(0, CHUNK)
            def _(e):
                ws = plsc.load_gather(
                    w_v, [jnp.full((16,), b, jnp.int32),
                          jnp.full((16,), e, jnp.int32)]
                )
                for j in range(D // 16):
                    sl = (e, pl.ds(j * 16, 16))
                    rows[sl] = rows[sl] * ws

            pltpu.sync_copy(rows, acc.at[dst_v.at[b]], add=True)

        def half(base, nch, ci, b):
            g_wait(b)
            @pl.when(ci + 2 < nch)
            def _():
                src_issue(base, ci + 2, b)
            dw_wait(b)
            scale_and_scatter(b)
            @pl.when(ci + 2 < nch)
            def _():
                dw_issue(base, ci + 2, b)
                src_wait(b)
                g_issue(b)

        def run_core(base, nch):
            src_issue(base, 0, 0)
            src_issue(base, 1, 1)
            dw_issue(base, 0, 0)
            dw_issue(base, 1, 1)
            src_wait(0)
            g_issue(0)
            src_wait(1)
            g_issue(1)

            @pl.loop(0, nch, step=2)
            def _(ci):
                half(base, nch, ci, 0)
                half(base, nch, ci + 1, 1)

        @pl.when(cid == 0)
        def _():
            run_core(sid * EPT0, NCH0)

        @pl.when(cid == 1)
        def _():
            run_core(SC0_TOT + sid * EPT1, NCH1)

        plsc.subcore_barrier()

        @pl.when(cid == 0)
        def _():
            pltpu.sync_copy(acc.at[pl.ds(row0, RPT)], out0.at[pl.ds(row0, RPT)])

        @pl.when(cid == 1)
        def _():
            pltpu.sync_copy(acc.at[pl.ds(row0, RPT)], out1.at[pl.ds(row0, RPT)])

    return k(h, src, dst, w)


def kernel(x, edge_index, edge_weight, W_ll, b_ll, W_self, b_self):
    h = _linear(x, W_ll, b_ll)
    pad = E_PAD - N_EDGES
    src = jnp.concatenate(
        [edge_index[0].astype(jnp.int32), jnp.arange(pad, dtype=jnp.int32)]
    )
    dst = jnp.concatenate(
        [edge_index[1].astype(jnp.int32), jnp.arange(pad, dtype=jnp.int32)]
    )
    w = jnp.pad(edge_weight.astype(jnp.float32), (0, pad))
    p0, p1 = _spmm_sc(h, src, dst, w)
    return _add(p0, p1)
